# baseline (speedup 1.0000x reference)
import jax
import jax.numpy as jnp
from jax import lax
from jax.experimental import pallas as pl

_N = 4096
_NPOINT = 512
_R2 = 0.25
_BIG = 1e10
_PREC = lax.Precision.DEFAULT
_TB = 128
_B = 2
_F = 8
_NL = _N // _F


def _body(x_ref, xt_ref, p_ref, pt_ref, ptf_ref, aafc_ref, aaff_ref,
          aapc_ref, aapr_ref, out_ref):
    f32 = jnp.float32
    i32 = jnp.int32
    iota_f = (lax.broadcasted_iota(i32, (_F, _NL), 0) * _NL
              + lax.broadcasted_iota(i32, (_F, _NL), 1))
    iota_a = (lax.broadcasted_iota(i32, (_F, 2 * _NPOINT // _F), 0)
              * (2 * _NPOINT // _F)
              + lax.broadcasted_iota(i32, (_F, 2 * _NPOINT // _F), 1))

    weights = []
    for b in range(_B):
        aapr = aapr_ref[b]
        cnt = jnp.zeros((1, _N), f32)
        for j in range(_N // _TB):
            pblk = p_ref[b, pl.ds(j * _TB, _TB), :]
            aac = aapc_ref[b, pl.ds(j * _TB, _TB), :]
            ab = jnp.dot(pblk, pt_ref[b], precision=_PREC)
            sq = (aac + aapr) - 2.0 * ab
            cnt = cnt + jnp.sum((sq < _R2).astype(f32), axis=0, keepdims=True)
        weights.append((1.0 / cnt).reshape(_F, _NL))

    def argmax_first(v):
        mx = jnp.max(v)
        return jnp.min(jnp.where(v == mx, iota_f, _N)).astype(i32)

    def step(t, carry):
        mf, lf, md, ld, acc = carry
        nmf, nlf, nmd, nld, nacc = [], [], [], [], []
        for b in range(_B):
            xl = x_ref[b, pl.ds(lf[b], 1), :]
            aal = aafc_ref[b, pl.ds(lf[b], 1), :]
            ab = jnp.concatenate(
                [jnp.dot(xl, xt_ref[b, :, pl.ds(k * _NL, _NL)],
                         precision=_PREC)
                 for k in range(_F)], axis=0)
            d = (aaff_ref[b] + aal) - 2.0 * ab
            mfb = jnp.minimum(mf[b], d)
            nxf = argmax_first(mfb)
            lp = p_ref[b, pl.ds(ld[b], 1), :]
            a0 = jnp.abs(ptf_ref[b, 0:_F, :] - lp[:, 0:1])
            a1 = jnp.abs(ptf_ref[b, _F:2 * _F, :] - lp[:, 1:2])
            a2 = jnp.abs(ptf_ref[b, 2 * _F:3 * _F, :] - lp[:, 2:3])
            dm = (a0 + a1) + a2
            mdb = jnp.minimum(md[b], dm)
            nxd = argmax_first(mdb * weights[b])
            accb = jnp.where(iota_a == (t + 1), nxf, acc[b])
            accb = jnp.where(iota_a == (_NPOINT + t + 1), nxd, accb)
            nmf.append(mfb); nlf.append(nxf)
            nmd.append(mdb); nld.append(nxd)
            nacc.append(accb)
        return nmf, nlf, nmd, nld, nacc

    mind0 = jnp.full((_F, _NL), _BIG, f32)
    acc0 = jnp.zeros((_F, 2 * _NPOINT // _F), i32)
    zero = jnp.int32(0)
    init = ([mind0] * _B, [zero] * _B, [mind0] * _B, [zero] * _B,
            [acc0] * _B)
    _, _, _, _, acc = lax.fori_loop(0, _NPOINT - 1, step, init)
    for b in range(_B):
        out_ref[b] = acc[b].reshape(1, 2 * _NPOINT)


def kernel(points, features, npoint):
    B, N, _ = points.shape
    f32 = jnp.float32
    ffps = jnp.concatenate([points, jnp.swapaxes(features, 1, 2)], axis=2)
    aaf = jnp.sum(ffps * ffps, axis=-1, keepdims=True)
    aap = jnp.sum(points * points, axis=-1, keepdims=True)
    xt = jnp.swapaxes(ffps, 1, 2)
    pt = jnp.swapaxes(points, 1, 2)
    ptf = pt.reshape(B, 3 * _F, _NL)
    aaf_fold = aaf.reshape(B, _F, _NL)
    aap_row = jnp.swapaxes(aap, 1, 2)

    out = pl.pallas_call(
        _body,
        out_shape=jax.ShapeDtypeStruct((B, 1, 2 * _NPOINT), jnp.int32),
    )(ffps.astype(f32), xt.astype(f32), points.astype(f32), pt.astype(f32),
      ptf.astype(f32), aaf.astype(f32), aaf_fold.astype(f32),
      aap.astype(f32), aap_row.astype(f32))
    return out.reshape(B, 2 * _NPOINT)


# speedup vs baseline: 1.0492x; 1.0492x over previous
import jax
import jax.numpy as jnp
from jax import lax
from jax.experimental import pallas as pl

_N = 4096
_NPOINT = 512
_R2 = 0.25
_BIG = 1e10
_PREC = lax.Precision.DEFAULT
_TB = 128
_B = 2
_F = 8
_NL = _N // _F


def _body(x_ref, xt_ref, p_ref, pt_ref, ptf_ref, aafc_ref, aaff_ref,
          aapc_ref, aapr_ref, out_ref):
    f32 = jnp.float32
    i32 = jnp.int32
    iota_f = (lax.broadcasted_iota(i32, (_F, _NL), 0) * _NL
              + lax.broadcasted_iota(i32, (_F, _NL), 1))
    iota_a = (lax.broadcasted_iota(i32, (_F, 2 * _NPOINT // _F), 0)
              * (2 * _NPOINT // _F)
              + lax.broadcasted_iota(i32, (_F, 2 * _NPOINT // _F), 1))

    weights = []
    for b in range(_B):
        aapr = aapr_ref[b]
        cnt = jnp.zeros((1, _N), f32)
        for j in range(_N // _TB):
            pblk = p_ref[b, pl.ds(j * _TB, _TB), :]
            aac = aapc_ref[b, pl.ds(j * _TB, _TB), :]
            ab = jnp.dot(pblk, pt_ref[b], precision=_PREC)
            sq = (aac + aapr) - 2.0 * ab
            cnt = cnt + jnp.sum((sq < _R2).astype(f32), axis=0, keepdims=True)
        weights.append((1.0 / cnt).reshape(_F, _NL))

    iota_r = lax.broadcasted_iota(i32, (1, _N), 1)

    def argmax_first_f(v):
        mx = jnp.max(v)
        return jnp.min(jnp.where(v == mx, iota_f, _N)).astype(i32)

    def argmax_first_r(v):
        mx = jnp.max(v)
        return jnp.min(jnp.where(v == mx, iota_r, _N)).astype(i32)

    def step(t, carry):
        mf, lf, md, ld, acc = carry
        nmf, nlf, nmd, nld, nacc = [], [], [], [], []
        for b in range(_B):
            xl = x_ref[b, pl.ds(lf[b], 1), :]
            aal = aafc_ref[b, pl.ds(lf[b], 1), :]
            ab = jnp.dot(xl, xt_ref[b], precision=_PREC)
            d = (aaff_ref[b] + aal) - 2.0 * ab
            mfb = jnp.minimum(mf[b], d)
            nxf = argmax_first_r(mfb)
            lp = p_ref[b, pl.ds(ld[b], 1), :]
            a0 = jnp.abs(ptf_ref[b, 0:_F, :] - lp[:, 0:1])
            a1 = jnp.abs(ptf_ref[b, _F:2 * _F, :] - lp[:, 1:2])
            a2 = jnp.abs(ptf_ref[b, 2 * _F:3 * _F, :] - lp[:, 2:3])
            dm = (a0 + a1) + a2
            mdb = jnp.minimum(md[b], dm)
            nxd = argmax_first_f(mdb * weights[b])
            accb = jnp.where(iota_a == (t + 1), nxf, acc[b])
            accb = jnp.where(iota_a == (_NPOINT + t + 1), nxd, accb)
            nmf.append(mfb); nlf.append(nxf)
            nmd.append(mdb); nld.append(nxd)
            nacc.append(accb)
        return nmf, nlf, nmd, nld, nacc

    mindf0 = jnp.full((1, _N), _BIG, f32)
    mindd0 = jnp.full((_F, _NL), _BIG, f32)
    acc0 = jnp.zeros((_F, 2 * _NPOINT // _F), i32)
    zero = jnp.int32(0)
    init = ([mindf0] * _B, [zero] * _B, [mindd0] * _B, [zero] * _B,
            [acc0] * _B)
    _, _, _, _, acc = lax.fori_loop(0, _NPOINT - 1, step, init)
    for b in range(_B):
        out_ref[b] = acc[b].reshape(1, 2 * _NPOINT)


def kernel(points, features, npoint):
    B, N, _ = points.shape
    f32 = jnp.float32
    ffps = jnp.concatenate([points, jnp.swapaxes(features, 1, 2)], axis=2)
    aaf = jnp.sum(ffps * ffps, axis=-1, keepdims=True)
    aap = jnp.sum(points * points, axis=-1, keepdims=True)
    xt = jnp.swapaxes(ffps, 1, 2)
    pt = jnp.swapaxes(points, 1, 2)
    ptf = pt.reshape(B, 3 * _F, _NL)
    aaf_row = jnp.swapaxes(aaf, 1, 2)
    aap_row = jnp.swapaxes(aap, 1, 2)

    out = pl.pallas_call(
        _body,
        out_shape=jax.ShapeDtypeStruct((B, 1, 2 * _NPOINT), jnp.int32),
    )(ffps.astype(f32), xt.astype(f32), points.astype(f32), pt.astype(f32),
      ptf.astype(f32), aaf.astype(f32), aaf_row.astype(f32),
      aap.astype(f32), aap_row.astype(f32))
    return out.reshape(B, 2 * _NPOINT)


# speedup vs baseline: 1.7990x; 1.7146x over previous
import functools

import jax
import jax.numpy as jnp
from jax import lax
from jax.experimental import pallas as pl
from jax.experimental.pallas import tpu as pltpu
from jax.experimental.pallas import tpu_sc as plsc

_N = 4096
_NPOINT = 512
_R2 = 0.25
_BIG = 1e10
_PREC = lax.Precision.DEFAULT
_TB = 128
_B = 2
_NSUB = 16
_CH = _N // _NSUB
_L = 16
_OPS = _NPOINT // _NSUB


def _density_body(p_ref, pt_ref, aapc_ref, aapr_ref, w_ref):
    f32 = jnp.float32
    for b in range(_B):
        pt = pt_ref[b]
        aapr = aapr_ref[b]
        cnt = jnp.zeros((1, _N), f32)
        for j in range(_N // _TB):
            pblk = p_ref[b, pl.ds(j * _TB, _TB), :]
            aac = aapc_ref[b, pl.ds(j * _TB, _TB), :]
            ab = jnp.dot(pblk, pt, precision=_PREC)
            sq = (aac + aapr) - 2.0 * ab
            cnt = cnt + jnp.sum((sq < _R2).astype(f32), axis=0, keepdims=True)
        w_ref[b] = 1.0 / cnt


def _ffps_body(x_ref, xt_ref, aafc_ref, aafr_ref, out_ref):
    f32 = jnp.float32
    i32 = jnp.int32
    iota = lax.broadcasted_iota(i32, (1, _N), 1)
    iota_out = lax.broadcasted_iota(i32, (1, _NPOINT), 1)

    def argmax_first(v):
        mx = jnp.max(v)
        return jnp.min(jnp.where(v == mx, iota, _N)).astype(i32)

    def step(t, carry):
        mf, lf, acc = carry
        nmf, nlf, nacc = [], [], []
        for b in range(_B):
            xl = x_ref[b, pl.ds(lf[b], 1), :]
            aal = aafc_ref[b, pl.ds(lf[b], 1), :]
            ab = jnp.dot(xl, xt_ref[b], precision=_PREC)
            d = (aafr_ref[b] + aal) - 2.0 * ab
            mfb = jnp.minimum(mf[b], d)
            nxf = argmax_first(mfb)
            nmf.append(mfb)
            nlf.append(nxf)
            nacc.append(jnp.where(iota_out == (t + 1), nxf, acc[b]))
        return nmf, nlf, nacc

    mind0 = jnp.full((1, _N), _BIG, f32)
    acc0 = jnp.zeros((1, _NPOINT), i32)
    zero = jnp.int32(0)
    _, _, acc = lax.fori_loop(0, _NPOINT - 1, step,
                              ([mind0] * _B, [zero] * _B, [acc0] * _B))
    for b in range(_B):
        out_ref[b] = acc[b]


def _dfps_sc_body(pt_hbm, w_hbm, out_hbm, px_v, py_v, pz_v, wv_v, mind_v,
                  pub_v, gat_v, tmp_v, myout_v, shared, shared_out):
    f32 = jnp.float32
    i32 = jnp.int32
    cid = lax.axis_index("c")
    sid = lax.axis_index("s")
    base = sid * _CH
    cbase = cid * (3 * _N)
    iota = lax.broadcasted_iota(i32, (_L,), 0)
    tz = jnp.minimum(sid, 0)

    def splat_i(x):
        return jnp.zeros((_L,), i32) + x + tz

    pltpu.sync_copy(pt_hbm.at[pl.ds(cbase + 0 * _N + base, _CH)],
                    px_v.at[pl.ds(8, _CH)])
    pltpu.sync_copy(pt_hbm.at[pl.ds(cbase + 1 * _N + base, _CH)],
                    py_v.at[pl.ds(8, _CH)])
    pltpu.sync_copy(pt_hbm.at[pl.ds(cbase + 2 * _N + base, _CH)],
                    pz_v.at[pl.ds(8, _CH)])
    pltpu.sync_copy(w_hbm.at[pl.ds(cid * _N + base, _CH)],
                    wv_v.at[pl.ds(8, _CH)])
    for v in range(_CH // _L):
        mind_v[pl.ds(v * _L, _L)] = jnp.full((_L,), _BIG, f32)
    lp0 = []
    for c in range(3):
        pltpu.sync_copy(pt_hbm.at[pl.ds(cbase + c * _N, _L)],
                        tmp_v.at[pl.ds(_L, _L)])
        lp0.append(plsc.load_gather(tmp_v, [splat_i(_L)]))
    lpx0, lpy0, lpz0 = lp0

    def step(t, carry):
        lpx, lpy, lpz, o0, o1 = carry
        bestv = jnp.full((_L,), -1.0, f32)
        besti = jnp.zeros((_L,), i32)
        for v in range(_CH // _L):
            sl = pl.ds(v * _L, _L)
            sl8 = pl.ds(8 + v * _L, _L)
            d = (jnp.abs(px_v[sl8] - lpx)
                 + jnp.abs(py_v[sl8] - lpy)) + jnp.abs(pz_v[sl8] - lpz)
            m = jnp.minimum(mind_v[sl], d)
            mind_v[sl] = m
            prod = m * wv_v[sl8]
            upd = prod > bestv
            bestv = jnp.where(upd, prod, bestv)
            besti = jnp.where(upd, base + (v * _L) + iota, besti)
        mloc = jnp.max(bestv)
        iloc = jnp.min(jnp.where(bestv == mloc, besti, _N))
        offs = splat_i(8 + (iloc - base))
        cx = plsc.load_gather(px_v, [offs])
        cy = plsc.load_gather(py_v, [offs])
        cz = plsc.load_gather(pz_v, [offs])
        pub = jnp.where(iota == 1, mloc, 0.0)
        pub = jnp.where(iota == 2, iloc.astype(f32), pub)
        pub = jnp.where(iota == 3, cx, pub)
        pub = jnp.where(iota == 4, cy, pub)
        pub = jnp.where(iota == 5, cz, pub)
        pub_v[...] = pub
        half = (t % 2) * (_NSUB * _L)
        pltpu.sync_copy(pub_v, shared.at[pl.ds(half + sid * _L, _L)])
        plsc.subcore_barrier()
        pltpu.sync_copy(shared.at[pl.ds(half, _NSUB * _L)], gat_v)
        vals = plsc.load_gather(gat_v, [iota * _L + 1 + tz])
        idxs = plsc.load_gather(gat_v, [iota * _L + 2 + tz])
        gm = jnp.max(vals)
        win_f = jnp.min(jnp.where(vals == gm, idxs, float(_N)))
        rw = jnp.min(jnp.where((vals == gm) & (idxs == win_f), iota, _L))
        win = win_f.astype(i32)
        nlpx = plsc.load_gather(gat_v, [splat_i(rw * _L + 3)])
        nlpy = plsc.load_gather(gat_v, [splat_i(rw * _L + 4)])
        nlpz = plsc.load_gather(gat_v, [splat_i(rw * _L + 5)])
        pos = (t + 1) - sid * _OPS
        o0 = jnp.where(iota == pos, win, o0)
        o1 = jnp.where(iota == (pos - _L), win, o1)
        return nlpx, nlpy, nlpz, o0, o1

    zo = jnp.zeros((_L,), i32)
    _, _, _, o0, o1 = lax.fori_loop(
        0, _NPOINT - 1, step, (lpx0, lpy0, lpz0, zo, zo))
    myout_v[pl.ds(0, _L)] = o0
    myout_v[pl.ds(_L, _L)] = o1
    pltpu.sync_copy(myout_v, shared_out.at[pl.ds(sid * 2 * _L, 2 * _L)])
    plsc.subcore_barrier()

    @pl.when(sid == 0)
    def _():
        pltpu.sync_copy(shared_out, out_hbm.at[pl.ds(cid * _NPOINT, _NPOINT)])


def _dfps_sc(pt, w):
    f32 = jnp.float32
    i32 = jnp.int32
    mesh = plsc.VectorSubcoreMesh(core_axis_name="c", subcore_axis_name="s")
    kern = functools.partial(
        pl.kernel,
        mesh=mesh,
        compiler_params=pltpu.CompilerParams(needs_layout_passes=False),
        out_type=jax.ShapeDtypeStruct((_B * _NPOINT,), i32),
        scratch_types=[
            pltpu.VMEM((_CH + 16,), f32),
            pltpu.VMEM((_CH + 16,), f32),
            pltpu.VMEM((_CH + 16,), f32),
            pltpu.VMEM((_CH + 16,), f32),
            pltpu.VMEM((_CH,), f32),
            pltpu.VMEM((_L,), f32),
            pltpu.VMEM((_NSUB * _L,), f32),
            pltpu.VMEM((2 * _L,), f32),
            pltpu.VMEM((2 * _L,), i32),
            pltpu.VMEM_SHARED((2 * _NSUB * _L,), f32),
            pltpu.VMEM_SHARED((_NSUB * 2 * _L,), i32),
        ],
    )(_dfps_sc_body)
    return kern(pt.reshape(-1), w.reshape(-1)).reshape(_B, _NPOINT)


def kernel(points, features, npoint):
    B, N, _ = points.shape
    f32 = jnp.float32
    ffps = jnp.concatenate([points, jnp.swapaxes(features, 1, 2)], axis=2)
    aaf = jnp.sum(ffps * ffps, axis=-1, keepdims=True)
    aap = jnp.sum(points * points, axis=-1, keepdims=True)
    xt = jnp.swapaxes(ffps, 1, 2)
    pt = jnp.swapaxes(points, 1, 2)
    aaf_row = jnp.swapaxes(aaf, 1, 2)
    aap_row = jnp.swapaxes(aap, 1, 2)

    weight = pl.pallas_call(
        _density_body,
        out_shape=jax.ShapeDtypeStruct((B, 1, _N), f32),
    )(points.astype(f32), pt.astype(f32), aap.astype(f32),
      aap_row.astype(f32))

    idx_d = _dfps_sc(pt.astype(f32), weight.reshape(B, _N))

    idx_f = pl.pallas_call(
        _ffps_body,
        out_shape=jax.ShapeDtypeStruct((B, 1, _NPOINT), jnp.int32),
    )(ffps.astype(f32), xt.astype(f32), aaf.astype(f32),
      aaf_row.astype(f32)).reshape(B, _NPOINT)

    return jnp.concatenate([idx_f, idx_d], axis=1)


# speedup vs baseline: 1.8089x; 1.0055x over previous
import functools

import jax
import jax.numpy as jnp
from jax import lax
from jax.experimental import pallas as pl
from jax.experimental.pallas import tpu as pltpu
from jax.experimental.pallas import tpu_sc as plsc

_N = 4096
_NPOINT = 512
_R2 = 0.25
_BIG = 1e10
_PREC = lax.Precision.DEFAULT
_TB = 128
_B = 2
_NSUB = 16
_CH = _N // _NSUB
_L = 16
_OPS = _NPOINT // _NSUB


def _density_body(p_ref, pt_ref, aapc_ref, aapr_ref, w_ref):
    f32 = jnp.float32
    for b in range(_B):
        pt = pt_ref[b]
        aapr = aapr_ref[b]
        cnt = jnp.zeros((1, _N), f32)
        for j in range(_N // _TB):
            pblk = p_ref[b, pl.ds(j * _TB, _TB), :]
            aac = aapc_ref[b, pl.ds(j * _TB, _TB), :]
            ab = jnp.dot(pblk, pt, precision=_PREC)
            sq = (aac + aapr) - 2.0 * ab
            cnt = cnt + jnp.sum((sq < _R2).astype(f32), axis=0, keepdims=True)
        w_ref[b] = 1.0 / cnt


def _ffps_body(x_ref, xt_ref, aafc_ref, aafr_ref, out_ref):
    f32 = jnp.float32
    i32 = jnp.int32
    iota = lax.broadcasted_iota(i32, (1, _N), 1)
    iota_out = lax.broadcasted_iota(i32, (1, _NPOINT), 1)

    def argmax_first(v):
        mx = jnp.max(v)
        return jnp.min(jnp.where(v == mx, iota, _N)).astype(i32)

    def step(t, carry):
        mf, lf, acc = carry
        nmf, nlf, nacc = [], [], []
        for b in range(_B):
            xl = x_ref[b, pl.ds(lf[b], 1), :]
            aal = aafc_ref[b, pl.ds(lf[b], 1), :]
            ab = jnp.dot(xl, xt_ref[b], precision=_PREC)
            d = (aafr_ref[b] + aal) - 2.0 * ab
            mfb = jnp.minimum(mf[b], d)
            nxf = argmax_first(mfb)
            nmf.append(mfb)
            nlf.append(nxf)
            nacc.append(jnp.where(iota_out == (t + 1), nxf, acc[b]))
        return nmf, nlf, nacc

    mind0 = jnp.full((1, _N), _BIG, f32)
    acc0 = jnp.zeros((1, _NPOINT), i32)
    zero = jnp.int32(0)
    _, _, acc = lax.fori_loop(0, _NPOINT - 1, step,
                              ([mind0] * _B, [zero] * _B, [acc0] * _B))
    for b in range(_B):
        out_ref[b] = acc[b]


def _dfps_sc_body(pt_hbm, w_hbm, out_hbm, px_v, py_v, pz_v, wv_v, mind_v,
                  pub_v, gat_v, tmp_v, myout_v, shared, shared_out):
    f32 = jnp.float32
    i32 = jnp.int32
    cid = lax.axis_index("c")
    sid = lax.axis_index("s")
    base = sid * _CH
    cbase = cid * (3 * _N)
    iota = lax.broadcasted_iota(i32, (_L,), 0)
    tz = jnp.minimum(sid, 0)

    def splat_i(x):
        return jnp.zeros((_L,), i32) + x + tz

    pltpu.sync_copy(pt_hbm.at[pl.ds(cbase + 0 * _N + base, _CH)],
                    px_v.at[pl.ds(8, _CH)])
    pltpu.sync_copy(pt_hbm.at[pl.ds(cbase + 1 * _N + base, _CH)],
                    py_v.at[pl.ds(8, _CH)])
    pltpu.sync_copy(pt_hbm.at[pl.ds(cbase + 2 * _N + base, _CH)],
                    pz_v.at[pl.ds(8, _CH)])
    pltpu.sync_copy(w_hbm.at[pl.ds(cid * _N + base, _CH)],
                    wv_v.at[pl.ds(8, _CH)])
    for v in range(_CH // _L):
        mind_v[pl.ds(v * _L, _L)] = jnp.full((_L,), _BIG, f32)
    lp0 = []
    for c in range(3):
        pltpu.sync_copy(pt_hbm.at[pl.ds(cbase + c * _N, _L)],
                        tmp_v.at[pl.ds(_L, _L)])
        lp0.append(plsc.load_gather(tmp_v, [splat_i(_L)]))
    lpx0, lpy0, lpz0 = lp0

    def sortkey(x):
        b = plsc.bitcast(x, jnp.int32)
        b = jnp.where(x != x, jnp.int32(0x7FC00000), b)
        return b ^ ((b >> 31) & jnp.int32(0x7FFFFFFF))

    def step(t, carry):
        lpx, lpy, lpz, o0, o1 = carry
        bestk = jnp.full((_L,), jnp.iinfo(jnp.int32).min, i32)
        besti = jnp.zeros((_L,), i32)
        for v in range(_CH // _L):
            sl = pl.ds(v * _L, _L)
            sl8 = pl.ds(8 + v * _L, _L)
            d = (jnp.abs(px_v[sl8] - lpx)
                 + jnp.abs(py_v[sl8] - lpy)) + jnp.abs(pz_v[sl8] - lpz)
            m = jnp.minimum(mind_v[sl], d)
            mind_v[sl] = m
            key = sortkey(m * wv_v[sl8])
            upd = key > bestk
            bestk = jnp.where(upd, key, bestk)
            besti = jnp.where(upd, base + (v * _L) + iota, besti)
        mk = jnp.max(bestk)
        iloc = jnp.min(jnp.where(bestk == mk, besti, _N))
        mkf = plsc.bitcast(jnp.zeros((_L,), i32) + mk, f32)
        offs = splat_i(8 + (iloc - base))
        cx = plsc.load_gather(px_v, [offs])
        cy = plsc.load_gather(py_v, [offs])
        cz = plsc.load_gather(pz_v, [offs])
        pub = jnp.where(iota == 1, mkf, 0.0)
        pub = jnp.where(iota == 2, iloc.astype(f32), pub)
        pub = jnp.where(iota == 3, cx, pub)
        pub = jnp.where(iota == 4, cy, pub)
        pub = jnp.where(iota == 5, cz, pub)
        pub_v[...] = pub
        half = (t % 2) * (_NSUB * _L)
        pltpu.sync_copy(pub_v, shared.at[pl.ds(half + sid * _L, _L)])
        plsc.subcore_barrier()
        pltpu.sync_copy(shared.at[pl.ds(half, _NSUB * _L)], gat_v)
        vals = plsc.bitcast(plsc.load_gather(gat_v, [iota * _L + 1 + tz]),
                            jnp.int32)
        idxs = plsc.load_gather(gat_v, [iota * _L + 2 + tz])
        gm = jnp.max(vals)
        win_f = jnp.min(jnp.where(vals == gm, idxs, float(_N)))
        rw = jnp.min(jnp.where((vals == gm) & (idxs == win_f), iota, _L))
        win = win_f.astype(i32)
        nlpx = plsc.load_gather(gat_v, [splat_i(rw * _L + 3)])
        nlpy = plsc.load_gather(gat_v, [splat_i(rw * _L + 4)])
        nlpz = plsc.load_gather(gat_v, [splat_i(rw * _L + 5)])
        pos = (t + 1) - sid * _OPS
        o0 = jnp.where(iota == pos, win, o0)
        o1 = jnp.where(iota == (pos - _L), win, o1)
        return nlpx, nlpy, nlpz, o0, o1

    zo = jnp.zeros((_L,), i32)
    _, _, _, o0, o1 = lax.fori_loop(
        0, _NPOINT - 1, step, (lpx0, lpy0, lpz0, zo, zo))
    myout_v[pl.ds(0, _L)] = o0
    myout_v[pl.ds(_L, _L)] = o1
    pltpu.sync_copy(myout_v, shared_out.at[pl.ds(sid * 2 * _L, 2 * _L)])
    plsc.subcore_barrier()

    @pl.when(sid == 0)
    def _():
        pltpu.sync_copy(shared_out, out_hbm.at[pl.ds(cid * _NPOINT, _NPOINT)])


def _dfps_sc(pt, w):
    f32 = jnp.float32
    i32 = jnp.int32
    mesh = plsc.VectorSubcoreMesh(core_axis_name="c", subcore_axis_name="s")
    kern = functools.partial(
        pl.kernel,
        mesh=mesh,
        compiler_params=pltpu.CompilerParams(needs_layout_passes=False),
        out_type=jax.ShapeDtypeStruct((_B * _NPOINT,), i32),
        scratch_types=[
            pltpu.VMEM((_CH + 16,), f32),
            pltpu.VMEM((_CH + 16,), f32),
            pltpu.VMEM((_CH + 16,), f32),
            pltpu.VMEM((_CH + 16,), f32),
            pltpu.VMEM((_CH,), f32),
            pltpu.VMEM((_L,), f32),
            pltpu.VMEM((_NSUB * _L,), f32),
            pltpu.VMEM((2 * _L,), f32),
            pltpu.VMEM((2 * _L,), i32),
            pltpu.VMEM_SHARED((2 * _NSUB * _L,), f32),
            pltpu.VMEM_SHARED((_NSUB * 2 * _L,), i32),
        ],
    )(_dfps_sc_body)
    return kern(pt.reshape(-1), w.reshape(-1)).reshape(_B, _NPOINT)


def kernel(points, features, npoint):
    B, N, _ = points.shape
    f32 = jnp.float32
    ffps = jnp.concatenate([points, jnp.swapaxes(features, 1, 2)], axis=2)
    aaf = jnp.sum(ffps * ffps, axis=-1, keepdims=True)
    aap = jnp.sum(points * points, axis=-1, keepdims=True)
    xt = jnp.swapaxes(ffps, 1, 2)
    pt = jnp.swapaxes(points, 1, 2)
    aaf_row = jnp.swapaxes(aaf, 1, 2)
    aap_row = jnp.swapaxes(aap, 1, 2)

    weight = pl.pallas_call(
        _density_body,
        out_shape=jax.ShapeDtypeStruct((B, 1, _N), f32),
    )(points.astype(f32), pt.astype(f32), aap.astype(f32),
      aap_row.astype(f32))

    idx_d = _dfps_sc(pt.astype(f32), weight.reshape(B, _N))

    idx_f = pl.pallas_call(
        _ffps_body,
        out_shape=jax.ShapeDtypeStruct((B, 1, _NPOINT), jnp.int32),
    )(ffps.astype(f32), xt.astype(f32), aaf.astype(f32),
      aaf_row.astype(f32)).reshape(B, _NPOINT)

    return jnp.concatenate([idx_f, idx_d], axis=1)
